# baseline (speedup 1.0000x reference)
import functools

import jax
import jax.numpy as jnp
from jax import lax
from jax.experimental import pallas as pl
from jax.experimental.pallas import tpu as pltpu
from jax.experimental.pallas import tpu_sc as plsc

_B = 16384
_CD = 64
_NW = 32
_ROWS = 2 * _B
_RPW = _ROWS // _NW
_CHUNK = 128
_NCHUNK = _RPW // _CHUNK
_PRED_PAD = 128


def _sc_gather(table, idx3):
    mesh = plsc.VectorSubcoreMesh(core_axis_name="c", subcore_axis_name="s")

    @functools.partial(
        pl.kernel,
        mesh=mesh,
        out_type=jax.ShapeDtypeStruct((_ROWS, _CD), jnp.float32),
        scratch_types=[
            pltpu.VMEM((_NCHUNK, _CHUNK), jnp.int32),
            pltpu.VMEM((_RPW, _CD), jnp.float32),
            pltpu.SemaphoreType.DMA,
        ],
        compiler_params=pltpu.CompilerParams(use_tc_tiling_on_sc=False),
    )
    def k(table_hbm, idx_hbm, out_hbm, idx_v, rows_v, sem):
        wid = lax.axis_index("s") * 2 + lax.axis_index("c")
        pltpu.sync_copy(idx_hbm.at[wid], idx_v)
        copies = []
        for j in range(_NCHUNK):
            copies.append(
                pltpu.async_copy(
                    table_hbm.at[idx_v.at[j]],
                    rows_v.at[pl.ds(j * _CHUNK, _CHUNK)],
                    sem,
                )
            )
        for c in copies:
            c.wait()
        pltpu.sync_copy(rows_v, out_hbm.at[pl.ds(wid * _RPW, _RPW)])

    return k(table, idx3)


def _tc_score(g2, pred_idx2, pred_pad, w_p, w_cc, bias):
    bb = 2048
    grid = _B // bb

    def body(g_ref, pi_ref, pt_ref, wp_ref, wcc_ref, b_ref, o_ref):
        p = jnp.dot(pt_ref[...], wp_ref[...], preferred_element_type=jnp.float32)
        onehot = (
            pi_ref[...] == lax.broadcasted_iota(jnp.int32, (bb, _PRED_PAD), 1)
        ).astype(jnp.float32)
        acc = (
            jnp.dot(g_ref[...], wcc_ref[...], preferred_element_type=jnp.float32)
            + jnp.dot(onehot, p, preferred_element_type=jnp.float32)
            + b_ref[...]
        )
        o_ref[...] = jnp.tanh(acc)

    return pl.pallas_call(
        body,
        grid=(grid,),
        in_specs=[
            pl.BlockSpec((bb, 2 * _CD), lambda i: (i, 0)),
            pl.BlockSpec((bb, 1), lambda i: (i, 0)),
            pl.BlockSpec((_PRED_PAD, _CD), lambda i: (0, 0)),
            pl.BlockSpec((_CD, _CD), lambda i: (0, 0)),
            pl.BlockSpec((2 * _CD, _CD), lambda i: (0, 0)),
            pl.BlockSpec((1, _CD), lambda i: (0, 0)),
        ],
        out_specs=pl.BlockSpec((bb, _CD), lambda i: (i, 0)),
        out_shape=jax.ShapeDtypeStruct((_B, _CD), jnp.float32),
    )(g2, pred_idx2, pred_pad, w_p, w_cc, bias)


def kernel(triplet_idx, predicate_idx, constant_table, predicate_table, W, b):
    idx3 = triplet_idx.astype(jnp.int32).reshape(_NW, _NCHUNK, _CHUNK)
    g = _sc_gather(constant_table, idx3)
    g2 = g.reshape(_B, 2 * _CD)
    pred_pad = jnp.zeros((_PRED_PAD, _CD), jnp.float32).at[
        : predicate_table.shape[0]
    ].set(predicate_table)
    pi2 = predicate_idx.astype(jnp.int32).reshape(_B, 1)
    return _tc_score(g2, pi2, pred_pad, W[:_CD], W[_CD:], b.reshape(1, _CD))
